# Initial kernel scaffold; baseline (speedup 1.0000x reference)
import jax
import jax.numpy as jnp
from jax.experimental import pallas as pl


def kernel(input_ids, table, gamma, beta):
    raise NotImplementedError("write your pallas kernel here")


# speedup vs baseline: 4.6383x; 4.6383x over previous
import functools

import jax
import jax.numpy as jnp
from jax import lax
from jax.experimental import pallas as pl
from jax.experimental.pallas import tpu as pltpu
from jax.experimental.pallas import tpu_sc as plsc

DIM = 32
LANES = 16
TOK = 20
NOUT = 4096 * 26
NC, NS = 2, 16
NW = NC * NS
OUT_PW = NOUT // NW
G = 16
C = G * TOK
NCHUNK = OUT_PW // G
DMA_SLICE = 80
EPS = 1e-12
RSQRT_MAGIC = 0x5F3759DF


_GDN = lax.GatherDimensionNumbers(
    offset_dims=(), collapsed_slice_dims=(0,), start_index_map=(0,))


def _lane_perm(v, idx):
    return lax.gather(v, idx.reshape(LANES, 1), _GDN, (1,),
                      mode=lax.GatherScatterMode.PROMISE_IN_BOUNDS)


def _lane_sum_bcast(v, perm_idx):
    for idx in perm_idx:
        v = v + _lane_perm(v, idx)
    return v


def _newton_rsqrt(x):
    i = plsc.bitcast(x, jnp.int32)
    i = RSQRT_MAGIC - lax.shift_right_logical(i, 1)
    y = plsc.bitcast(i, jnp.float32)
    xh = x * 0.5
    y = y * (1.5 - xh * y * y)
    y = y * (1.5 - xh * y * y)
    return y


def _make_sc_kernel():
    mesh = plsc.VectorSubcoreMesh(core_axis_name="c", subcore_axis_name="s")

    @functools.partial(
        pl.kernel,
        out_type=jax.ShapeDtypeStruct((NOUT, DIM), jnp.float32),
        mesh=mesh,
        compiler_params=pltpu.CompilerParams(
            needs_layout_passes=False, use_tc_tiling_on_sc=False),
        scratch_types=[
            pltpu.VMEM((C,), jnp.int32),
            pltpu.VMEM((C, DIM), jnp.float32),
            pltpu.VMEM((G, DIM), jnp.float32),
            pltpu.VMEM((DIM,), jnp.float32),
            pltpu.VMEM((DIM,), jnp.float32),
            pltpu.SemaphoreType.DMA,
        ],
    )
    def sc_kernel(ids_hbm, table_hbm, gamma_hbm, beta_hbm, out_hbm,
                  idx_v, rows_v, out_v, gam_v, bet_v, sem):
        wid = lax.axis_index("s") * NC + lax.axis_index("c")
        lane = lax.iota(jnp.int32, LANES)
        perm_idx = [lane ^ k for k in (1, 2, 4, 8)]
        pltpu.sync_copy(gamma_hbm, gam_v)
        pltpu.sync_copy(beta_hbm, bet_v)
        glo = gam_v[pl.ds(0, LANES)]
        ghi = gam_v[pl.ds(LANES, LANES)]
        blo = bet_v[pl.ds(0, LANES)] * float(TOK)
        bhi = bet_v[pl.ds(LANES, LANES)] * float(TOK)
        base_out = wid * OUT_PW

        def chunk(c, carry):
            orow = base_out + c * G
            pltpu.sync_copy(ids_hbm.at[pl.ds(orow * TOK, C)], idx_v)
            copies = []
            for j in range(C // DMA_SLICE):
                copies.append(pltpu.async_copy(
                    table_hbm.at[idx_v.at[pl.ds(j * DMA_SLICE, DMA_SLICE)]],
                    rows_v.at[pl.ds(j * DMA_SLICE, DMA_SLICE)],
                    sem,
                ))
            for cp in copies:
                cp.wait()

            def group(g, gcarry):
                r0 = g * TOK
                acc_lo = jnp.zeros((LANES,), jnp.float32)
                acc_hi = jnp.zeros((LANES,), jnp.float32)
                for l in range(TOK):
                    lo = rows_v[r0 + l, pl.ds(0, LANES)]
                    hi = rows_v[r0 + l, pl.ds(LANES, LANES)]
                    tot = _lane_sum_bcast(lo + hi, perm_idx)
                    tot2 = _lane_sum_bcast(lo * lo + hi * hi, perm_idx)
                    mean = tot * (1.0 / DIM)
                    var = tot2 * (1.0 / DIM) - mean * mean
                    inv = _newton_rsqrt(var + EPS)
                    acc_lo = acc_lo + (lo - mean) * inv
                    acc_hi = acc_hi + (hi - mean) * inv
                out_v[g, pl.ds(0, LANES)] = acc_lo * glo + blo
                out_v[g, pl.ds(LANES, LANES)] = acc_hi * ghi + bhi
                return gcarry

            lax.fori_loop(0, G, group, 0)
            pltpu.sync_copy(out_v, out_hbm.at[pl.ds(orow, G)])
            return carry

        lax.fori_loop(0, NCHUNK, chunk, 0)

    return sc_kernel


_SC_KERNEL = _make_sc_kernel()


def kernel(input_ids, table, gamma, beta):
    b, f, t = input_ids.shape
    ids_flat = input_ids.reshape(-1)
    out = _SC_KERNEL(ids_flat, table, gamma, beta)
    return out.reshape(b, f, DIM)


# speedup vs baseline: 7.2071x; 1.5538x over previous
import functools

import jax
import jax.numpy as jnp
from jax import lax
from jax.experimental import pallas as pl
from jax.experimental.pallas import tpu as pltpu
from jax.experimental.pallas import tpu_sc as plsc

DIM = 32
LANES = 16
TOK = 20
NOUT = 4096 * 26
NC, NS = 2, 16
NW = NC * NS
OUT_PW = NOUT // NW
IDX_PW = OUT_PW * TOK
G = 32
C = G * TOK
NCHUNK = OUT_PW // G
DMA_SLICE = 128
NSLICE = C // DMA_SLICE
EPS = 1e-12
RSQRT_MAGIC = 0x5F3759DF

_GDN = lax.GatherDimensionNumbers(
    offset_dims=(), collapsed_slice_dims=(0,), start_index_map=(0,))


def _lane_perm(v, idx):
    return lax.gather(v, idx.reshape(LANES, 1), _GDN, (1,),
                      mode=lax.GatherScatterMode.PROMISE_IN_BOUNDS)


def _newton_rsqrt(x):
    i = plsc.bitcast(x, jnp.int32)
    i = RSQRT_MAGIC - lax.shift_right_logical(i, 1)
    y = plsc.bitcast(i, jnp.float32)
    y = y * (1.5 - (x * 0.5) * y * y)
    return y


def _make_sc_kernel():
    mesh = plsc.VectorSubcoreMesh(core_axis_name="c", subcore_axis_name="s")

    @functools.partial(
        pl.kernel,
        out_type=jax.ShapeDtypeStruct((NOUT, DIM), jnp.float32),
        mesh=mesh,
        compiler_params=pltpu.CompilerParams(
            needs_layout_passes=False, use_tc_tiling_on_sc=False),
        scratch_types=[
            pltpu.VMEM((IDX_PW,), jnp.int32),
            pltpu.VMEM((C, DIM), jnp.float32),
            pltpu.VMEM((C, DIM), jnp.float32),
            pltpu.VMEM((G, DIM), jnp.float32),
            pltpu.VMEM((G, DIM), jnp.float32),
            pltpu.VMEM((DIM,), jnp.float32),
            pltpu.VMEM((DIM,), jnp.float32),
            pltpu.SemaphoreType.DMA,
            pltpu.SemaphoreType.DMA,
            pltpu.SemaphoreType.DMA,
        ],
    )
    def sc_kernel(ids_hbm, table_hbm, gamma_hbm, beta_hbm, out_hbm,
                  idx_all, rows0, rows1, out0, out1, gam_v, bet_v,
                  rsem, osem0, osem1):
        wid = lax.axis_index("s") * NC + lax.axis_index("c")
        base_out = wid * OUT_PW
        pltpu.sync_copy(ids_hbm.at[pl.ds(base_out * TOK, IDX_PW)], idx_all)
        pltpu.sync_copy(gamma_hbm, gam_v)
        pltpu.sync_copy(beta_hbm, bet_v)
        glo = gam_v[pl.ds(0, LANES)]
        ghi = gam_v[pl.ds(LANES, LANES)]
        blo = bet_v[pl.ds(0, LANES)] * float(TOK)
        bhi = bet_v[pl.ds(LANES, LANES)] * float(TOK)
        idx15 = jnp.full((LANES,), LANES - 1, jnp.int32)

        rows = (rows0, rows1)
        outs = (out0, out1)
        osems = (osem0, osem1)

        def row_copies(c, b):
            return [
                pltpu.make_async_copy(
                    table_hbm.at[idx_all.at[
                        pl.ds(c * C + j * DMA_SLICE, DMA_SLICE)]],
                    rows[b].at[pl.ds(j * DMA_SLICE, DMA_SLICE)],
                    rsem,
                )
                for j in range(NSLICE)
            ]

        def out_copy(c, b):
            return pltpu.make_async_copy(
                outs[b], out_hbm.at[pl.ds(base_out + c * G, G)], osems[b])

        def lane_total(v):
            return _lane_perm(jnp.cumsum(v), idx15)

        def compute(b):
            rv = rows[b]
            ov = outs[b]

            def group(g, gcarry):
                r0 = g * TOK
                acc_lo = jnp.zeros((LANES,), jnp.float32)
                acc_hi = jnp.zeros((LANES,), jnp.float32)
                for l in range(TOK):
                    lo = rv[r0 + l, pl.ds(0, LANES)]
                    hi = rv[r0 + l, pl.ds(LANES, LANES)]
                    tot = lane_total(lo + hi)
                    tot2 = lane_total(lo * lo + hi * hi)
                    mean = tot * (1.0 / DIM)
                    var = tot2 * (1.0 / DIM) - mean * mean
                    inv = _newton_rsqrt(var + EPS)
                    acc_lo = acc_lo + (lo - mean) * inv
                    acc_hi = acc_hi + (hi - mean) * inv
                ov[g, pl.ds(0, LANES)] = acc_lo * glo + blo
                ov[g, pl.ds(LANES, LANES)] = acc_hi * ghi + bhi
                return gcarry

            lax.fori_loop(0, G, group, 0)

        def pair(c2, carry):
            for b in (0, 1):
                c = c2 * 2 + b
                for cp in row_copies(c, b):
                    cp.wait()

                @pl.when(c + 1 < NCHUNK)
                def _():
                    for cp in row_copies(c + 1, b ^ 1):
                        cp.start()

                @pl.when(c >= 2)
                def _():
                    out_copy(c - 2, b).wait()

                compute(b)
                out_copy(c, b).start()
            return carry

        for cp in row_copies(0, 0):
            cp.start()
        lax.fori_loop(0, NCHUNK // 2, pair, 0)
        out_copy(NCHUNK - 2, 0).wait()
        out_copy(NCHUNK - 1, 1).wait()

    return sc_kernel


_SC_KERNEL = _make_sc_kernel()


def kernel(input_ids, table, gamma, beta):
    b, f, t = input_ids.shape
    ids_flat = input_ids.reshape(-1)
    out = _SC_KERNEL(ids_flat, table, gamma, beta)
    return out.reshape(b, f, DIM)
